# baseline (speedup 1.0000x reference)
import functools

import jax
import jax.numpy as jnp
from jax import lax
from jax.experimental import pallas as pl
from jax.experimental.pallas import tpu as pltpu
from jax.experimental.pallas import tpu_sc as plsc

BATCH = 16384
EMBED = 64
HID = 32

NUM_CORES = 2
NUM_SUBCORES = 16
NUM_WORKERS = NUM_CORES * NUM_SUBCORES
B_PER_W = BATCH // NUM_WORKERS

CHUNK = 128
NCHUNK = B_PER_W // CHUNK

ROW_BLOCK = 2048


def _gather_body(uids_hbm, iids_hbm, utab_hbm, itab_hbm, uout_hbm, iout_hbm,
                 uidx_v, iidx_v, ubuf0, ubuf1, ibuf0, ibuf1,
                 ugsem, igsem, uwsem, iwsem):
    wid = lax.axis_index("s") * NUM_CORES + lax.axis_index("c")
    base = wid * B_PER_W
    pltpu.sync_copy(uids_hbm.at[pl.ds(base, B_PER_W)], uidx_v)
    pltpu.sync_copy(iids_hbm.at[pl.ds(base, B_PER_W)], iidx_v)

    ubufs = (ubuf0, ubuf1)
    ibufs = (ibuf0, ibuf1)

    def gather(c, b):
        ucp = pltpu.async_copy(
            utab_hbm.at[uidx_v.at[pl.ds(c * CHUNK, CHUNK)]], ubufs[b], ugsem)
        icp = pltpu.async_copy(
            itab_hbm.at[iidx_v.at[pl.ds(c * CHUNK, CHUNK)]], ibufs[b], igsem)
        return ucp, icp

    def write(c, b):
        dst = pl.ds(base + c * CHUNK, CHUNK)
        ucp = pltpu.async_copy(ubufs[b], uout_hbm.at[dst], uwsem)
        icp = pltpu.async_copy(ibufs[b], iout_hbm.at[dst], iwsem)
        return ucp, icp

    pend_g = gather(0, 0)
    pend_w = [None, None]
    for c in range(NCHUNK):
        b = c & 1
        pend_g[0].wait()
        pend_g[1].wait()
        if c + 1 < NCHUNK:
            if pend_w[b ^ 1] is not None:
                pend_w[b ^ 1][0].wait()
                pend_w[b ^ 1][1].wait()
            pend_g = gather(c + 1, b ^ 1)
        pend_w[b] = write(c, b)
    for pw in pend_w:
        if pw is not None:
            pw[0].wait()
            pw[1].wait()


_sc_gather = functools.partial(
    pl.kernel,
    out_type=[
        jax.ShapeDtypeStruct((BATCH, EMBED), jnp.float32),
        jax.ShapeDtypeStruct((BATCH, EMBED), jnp.float32),
    ],
    mesh=plsc.VectorSubcoreMesh(core_axis_name="c", subcore_axis_name="s"),
    compiler_params=pltpu.CompilerParams(use_tc_tiling_on_sc=False),
    scratch_types=[
        pltpu.VMEM((B_PER_W,), jnp.int32),
        pltpu.VMEM((B_PER_W,), jnp.int32),
        pltpu.VMEM((CHUNK, EMBED), jnp.float32),
        pltpu.VMEM((CHUNK, EMBED), jnp.float32),
        pltpu.VMEM((CHUNK, EMBED), jnp.float32),
        pltpu.VMEM((CHUNK, EMBED), jnp.float32),
        pltpu.SemaphoreType.DMA,
        pltpu.SemaphoreType.DMA,
        pltpu.SemaphoreType.DMA,
        pltpu.SemaphoreType.DMA,
    ],
)(_gather_body)


def _mlp_body(ue_ref, ie_ref, uW1_ref, ub1_ref, uW2_ref,
              ub2_ref, iW1_ref, ib1_ref, iW2_ref, ib2_ref, out_ref):
    ue = ue_ref[...]
    ie = ie_ref[...]
    uh = jnp.maximum(
        jnp.dot(ue, uW1_ref[...], preferred_element_type=jnp.float32)
        + ub1_ref[...], 0.0)
    uv = jnp.dot(uh, uW2_ref[...], preferred_element_type=jnp.float32) \
        + ub2_ref[...]
    ih = jnp.maximum(
        jnp.dot(ie, iW1_ref[...], preferred_element_type=jnp.float32)
        + ib1_ref[...], 0.0)
    iv = jnp.dot(ih, iW2_ref[...], preferred_element_type=jnp.float32) \
        + ib2_ref[...]
    un = jnp.sqrt(jnp.sum(uv * uv, axis=1))
    iN = jnp.sqrt(jnp.sum(iv * iv, axis=1))
    dot = jnp.sum(uv * iv, axis=1)
    eps = jnp.float32(1e-12)
    out_ref[...] = dot / (jnp.maximum(un, eps) * jnp.maximum(iN, eps))


def _mlp_call(ue, ie, uW1, ub1, uW2, ub2, iW1, ib1, iW2, ib2):
    n_blocks = BATCH // ROW_BLOCK
    w_spec = lambda shape: pl.BlockSpec(shape, lambda i: (0,) * len(shape))
    return pl.pallas_call(
        _mlp_body,
        grid=(n_blocks,),
        in_specs=[
            pl.BlockSpec((ROW_BLOCK, EMBED), lambda i: (i, 0)),
            pl.BlockSpec((ROW_BLOCK, EMBED), lambda i: (i, 0)),
            w_spec((EMBED, HID)),
            w_spec((1, HID)),
            w_spec((HID, HID)),
            w_spec((1, HID)),
            w_spec((EMBED, HID)),
            w_spec((1, HID)),
            w_spec((HID, HID)),
            w_spec((1, HID)),
        ],
        out_specs=pl.BlockSpec((ROW_BLOCK,), lambda i: (i,)),
        out_shape=jax.ShapeDtypeStruct((BATCH,), jnp.float32),
    )(ue, ie, uW1, ub1, uW2, ub2, iW1, ib1, iW2, ib2)


def kernel(user_ids, item_ids, user_table, item_table,
           uW1, ub1, uW2, ub2, iW1, ib1, iW2, ib2):
    ue, ie = _sc_gather(user_ids, item_ids, user_table, item_table)
    return _mlp_call(ue, ie,
                     uW1, ub1.reshape(1, HID), uW2, ub2.reshape(1, HID),
                     iW1, ib1.reshape(1, HID), iW2, ib2.reshape(1, HID))


# speedup vs baseline: 1.0077x; 1.0077x over previous
import functools

import jax
import jax.numpy as jnp
from jax import lax
from jax.experimental import pallas as pl
from jax.experimental.pallas import tpu as pltpu
from jax.experimental.pallas import tpu_sc as plsc

BATCH = 16384
EMBED = 64
HID = 32
PAIR = 2 * EMBED

NUM_CORES = 2
NUM_SUBCORES = 16
NUM_WORKERS = NUM_CORES * NUM_SUBCORES
B_PER_W = BATCH // NUM_WORKERS

CHUNK = 128
NCHUNK = B_PER_W // CHUNK

ROW_BLOCK = 2048


def _gather_body(uids_hbm, iids_hbm, utab_hbm, itab_hbm, uout_hbm, iout_hbm,
                 uidx_v, iidx_v, ubuf0, ubuf1, ibuf0, ibuf1,
                 ugsem, igsem, uwsem, iwsem):
    wid = lax.axis_index("s") * NUM_CORES + lax.axis_index("c")
    base = wid * B_PER_W
    pltpu.sync_copy(uids_hbm.at[pl.ds(base, B_PER_W)], uidx_v)
    pltpu.sync_copy(iids_hbm.at[pl.ds(base, B_PER_W)], iidx_v)

    ubufs = (ubuf0, ubuf1)
    ibufs = (ibuf0, ibuf1)

    def gather(c, b):
        ucp = pltpu.async_copy(
            utab_hbm.at[uidx_v.at[pl.ds(c * CHUNK, CHUNK)]], ubufs[b], ugsem)
        icp = pltpu.async_copy(
            itab_hbm.at[iidx_v.at[pl.ds(c * CHUNK, CHUNK)]], ibufs[b], igsem)
        return ucp, icp

    def write(c, b):
        dst = pl.ds(base + c * CHUNK, CHUNK)
        ucp = pltpu.async_copy(ubufs[b], uout_hbm.at[dst], uwsem)
        icp = pltpu.async_copy(ibufs[b], iout_hbm.at[dst], iwsem)
        return ucp, icp

    pend_g = gather(0, 0)
    pend_w = [None, None]
    for c in range(NCHUNK):
        b = c & 1
        pend_g[0].wait()
        pend_g[1].wait()
        if c + 1 < NCHUNK:
            if pend_w[b ^ 1] is not None:
                pend_w[b ^ 1][0].wait()
                pend_w[b ^ 1][1].wait()
            pend_g = gather(c + 1, b ^ 1)
        pend_w[b] = write(c, b)
    for pw in pend_w:
        if pw is not None:
            pw[0].wait()
            pw[1].wait()


_sc_gather = functools.partial(
    pl.kernel,
    out_type=[
        jax.ShapeDtypeStruct((BATCH, PAIR), jnp.float32),
        jax.ShapeDtypeStruct((BATCH, PAIR), jnp.float32),
    ],
    mesh=plsc.VectorSubcoreMesh(core_axis_name="c", subcore_axis_name="s"),
    compiler_params=pltpu.CompilerParams(use_tc_tiling_on_sc=True),
    scratch_types=[
        pltpu.VMEM((B_PER_W,), jnp.int32),
        pltpu.VMEM((B_PER_W,), jnp.int32),
        pltpu.VMEM((CHUNK, PAIR), jnp.float32),
        pltpu.VMEM((CHUNK, PAIR), jnp.float32),
        pltpu.VMEM((CHUNK, PAIR), jnp.float32),
        pltpu.VMEM((CHUNK, PAIR), jnp.float32),
        pltpu.SemaphoreType.DMA,
        pltpu.SemaphoreType.DMA,
        pltpu.SemaphoreType.DMA,
        pltpu.SemaphoreType.DMA,
    ],
)(_gather_body)


def _mlp_body(uid_ref, iid_ref, ue_ref, ie_ref, uW1_ref, ub1_ref, uW2_ref,
              ub2_ref, iW1_ref, ib1_ref, iW2_ref, ib2_ref, out_ref):
    uodd = (uid_ref[...] & 1).astype(jnp.float32)[:, None]
    iodd = (iid_ref[...] & 1).astype(jnp.float32)[:, None]
    up = ue_ref[...]
    ip = ie_ref[...]
    ue = up[:, :EMBED] * (1.0 - uodd) + up[:, EMBED:] * uodd
    ie = ip[:, :EMBED] * (1.0 - iodd) + ip[:, EMBED:] * iodd
    uh = jnp.maximum(
        jnp.dot(ue, uW1_ref[...], preferred_element_type=jnp.float32)
        + ub1_ref[...], 0.0)
    uv = jnp.dot(uh, uW2_ref[...], preferred_element_type=jnp.float32) \
        + ub2_ref[...]
    ih = jnp.maximum(
        jnp.dot(ie, iW1_ref[...], preferred_element_type=jnp.float32)
        + ib1_ref[...], 0.0)
    iv = jnp.dot(ih, iW2_ref[...], preferred_element_type=jnp.float32) \
        + ib2_ref[...]
    un = jnp.sqrt(jnp.sum(uv * uv, axis=1))
    iN = jnp.sqrt(jnp.sum(iv * iv, axis=1))
    dot = jnp.sum(uv * iv, axis=1)
    eps = jnp.float32(1e-12)
    out_ref[...] = dot / (jnp.maximum(un, eps) * jnp.maximum(iN, eps))


def _mlp_call(uid, iid, ue, ie, uW1, ub1, uW2, ub2, iW1, ib1, iW2, ib2):
    n_blocks = BATCH // ROW_BLOCK
    w_spec = lambda shape: pl.BlockSpec(shape, lambda i: (0,) * len(shape))
    return pl.pallas_call(
        _mlp_body,
        grid=(n_blocks,),
        in_specs=[
            pl.BlockSpec((ROW_BLOCK,), lambda i: (i,)),
            pl.BlockSpec((ROW_BLOCK,), lambda i: (i,)),
            pl.BlockSpec((ROW_BLOCK, PAIR), lambda i: (i, 0)),
            pl.BlockSpec((ROW_BLOCK, PAIR), lambda i: (i, 0)),
            w_spec((EMBED, HID)),
            w_spec((1, HID)),
            w_spec((HID, HID)),
            w_spec((1, HID)),
            w_spec((EMBED, HID)),
            w_spec((1, HID)),
            w_spec((HID, HID)),
            w_spec((1, HID)),
        ],
        out_specs=pl.BlockSpec((ROW_BLOCK,), lambda i: (i,)),
        out_shape=jax.ShapeDtypeStruct((BATCH,), jnp.float32),
    )(uid, iid, ue, ie, uW1, ub1, uW2, ub2, iW1, ib1, iW2, ib2)


def kernel(user_ids, item_ids, user_table, item_table,
           uW1, ub1, uW2, ub2, iW1, ib1, iW2, ib2):
    utab2 = user_table.reshape(-1, PAIR)
    itab2 = item_table.reshape(-1, PAIR)
    upair = lax.shift_right_logical(user_ids, 1)
    ipair = lax.shift_right_logical(item_ids, 1)
    ue, ie = _sc_gather(upair, ipair, utab2, itab2)
    return _mlp_call(user_ids, item_ids, ue, ie,
                     uW1, ub1.reshape(1, HID), uW2, ub2.reshape(1, HID),
                     iW1, ib1.reshape(1, HID), iW2, ib2.reshape(1, HID))
